# baseline (speedup 1.0000x reference)
import functools

import jax
import jax.numpy as jnp
from jax import lax
from jax.experimental import pallas as pl
from jax.experimental.pallas import tpu as pltpu
from jax.experimental.pallas import tpu_sc as plsc

_VOCAB = 1000000
_EMBED = 64
_B = 16384
_L = 50
_NC = 2
_NS = 16
_NW = _NC * _NS
_KB = 256
_KJ = _KB // 128
_CPL = _B // _KB
_UNITS = _L * _CPL
_UPT = _UNITS // _NW
_NBUF = 2


def _sc_gather(idx_t, table):
    mesh = plsc.VectorSubcoreMesh(core_axis_name="c", subcore_axis_name="s")

    @functools.partial(
        pl.kernel,
        out_type=jax.ShapeDtypeStruct((_L, _EMBED, _B), jnp.float32),
        mesh=mesh,
        scratch_types=[
            pltpu.VMEM((_NBUF, _KJ, 128), jnp.int32),
            pltpu.VMEM((_NBUF, _KJ, 128, _EMBED), jnp.float32),
            pltpu.VMEM((_NBUF, _EMBED, _KB), jnp.float32),
            pltpu.SemaphoreType.DMA,
            pltpu.SemaphoreType.DMA,
        ],
        compiler_params=pltpu.CompilerParams(
            use_tc_tiling_on_sc=False, needs_layout_passes=False
        ),
    )
    def k(idx_hbm, table_hbm, out_hbm, idx_v, rows_v, trans_v, sem0, sem1):
        sems = (sem0, sem1)
        wid = lax.axis_index("s") * _NC + lax.axis_index("c")
        u0 = wid * _UPT
        iota = lax.iota(jnp.int32, 16)
        dvecs = [jnp.full((16,), d, jnp.int32) for d in range(_EMBED)]

        def load_and_fire(u, b):
            l = u // _CPL
            b0 = (u % _CPL) * _KB
            for j in range(_KJ):
                pltpu.sync_copy(
                    idx_hbm.at[l, pl.ds(b0 + j * 128, 128)], idx_v.at[b].at[j]
                )
            for j in range(_KJ):
                pltpu.async_copy(
                    table_hbm.at[idx_v.at[b].at[j]], rows_v.at[b].at[j], sems[b]
                )

        def drain_transpose_store(u, b):
            for j in range(_KJ):
                pltpu.make_async_copy(
                    table_hbm.at[idx_v.at[b].at[j]], rows_v.at[b].at[j], sems[b]
                ).wait()

            @pl.loop(0, _KB // 16)
            def _t(w0):
                w = w0 * 16 + iota
                j_ids = lax.shift_right_logical(w, 7)
                r_ids = lax.bitwise_and(w, 127)
                for d in range(_EMBED):
                    v = plsc.load_gather(rows_v.at[b], [j_ids, r_ids, dvecs[d]])
                    trans_v.at[b][d, pl.ds(w0 * 16, 16)] = v

            l = u // _CPL
            b0 = (u % _CPL) * _KB
            pltpu.sync_copy(
                trans_v.at[b], out_hbm.at[l, slice(None), pl.ds(b0, _KB)]
            )

        for b in range(_NBUF):
            load_and_fire(u0 + b, b)

        @pl.loop(0, _UPT - _NBUF, step=_NBUF)
        def _unit(i):
            for b in range(_NBUF):
                drain_transpose_store(u0 + i + b, b)
                load_and_fire(u0 + i + b + _NBUF, b)

        for b in range(_NBUF):
            drain_transpose_store(u0 + _UPT - _NBUF + b, b)

    return k(idx_t, table)


def kernel(inputs, embeddings):
    idx_t = jnp.transpose(inputs.astype(jnp.int32))
    out_t = _sc_gather(idx_t, embeddings)
    return jnp.transpose(out_t, (2, 0, 1))


# speedup vs baseline: 1.7231x; 1.7231x over previous
import functools

import jax
import jax.numpy as jnp
from jax import lax
from jax.experimental import pallas as pl
from jax.experimental.pallas import tpu as pltpu
from jax.experimental.pallas import tpu_sc as plsc

_VOCAB = 1000000
_EMBED = 64
_B = 16384
_L = 50
_NC = 2
_NS = 16
_NW = _NC * _NS
_BPW = _B // _NW
_KC = 8
_NBUF = 2
_CHUNKS = _BPW // _KC


def _sc_gather(idx, table):
    mesh = plsc.VectorSubcoreMesh(core_axis_name="c", subcore_axis_name="s")

    @functools.partial(
        pl.kernel,
        out_type=jax.ShapeDtypeStruct((_B, _L, _EMBED), jnp.float32),
        mesh=mesh,
        scratch_types=[
            pltpu.VMEM((_NBUF, _KC, _L), jnp.int32),
            pltpu.VMEM((_NBUF, _KC, _L, _EMBED), jnp.float32),
            pltpu.SemaphoreType.DMA,
            pltpu.SemaphoreType.DMA,
        ],
        compiler_params=pltpu.CompilerParams(use_tc_tiling_on_sc=False),
    )
    def k(idx_hbm, table_hbm, out_hbm, idx_v, rows_v, sem0, sem1):
        sems = (sem0, sem1)
        wid = lax.axis_index("s") * _NC + lax.axis_index("c")
        base_b = wid * _BPW

        def load_and_fire(ci, b):
            r = base_b + ci * _KC
            pltpu.sync_copy(idx_hbm.at[pl.ds(r, _KC)], idx_v.at[b])
            for j in range(_KC):
                pltpu.async_copy(
                    table_hbm.at[idx_v.at[b].at[j]], rows_v.at[b].at[j], sems[b]
                )

        def drain_and_store(ci, b):
            for j in range(_KC):
                pltpu.make_async_copy(
                    table_hbm.at[idx_v.at[b].at[j]], rows_v.at[b].at[j], sems[b]
                ).wait()
            r = base_b + ci * _KC
            pltpu.sync_copy(rows_v.at[b], out_hbm.at[pl.ds(r, _KC)])

        for b in range(_NBUF):
            load_and_fire(b, b)

        @pl.loop(0, _CHUNKS - _NBUF, step=_NBUF)
        def _chunk(i):
            for b in range(_NBUF):
                drain_and_store(i + b, b)
                load_and_fire(i + b + _NBUF, b)

        for b in range(_NBUF):
            drain_and_store(_CHUNKS - _NBUF + b, b)

    return k(idx, table)


def kernel(inputs, embeddings):
    return _sc_gather(inputs.astype(jnp.int32), embeddings)
